# baseline (speedup 1.0000x reference)
import jax
import jax.numpy as jnp
from jax.experimental import pallas as pl
from jax.experimental.pallas import tpu as pltpu

_EPS = 1e-12
_BS = 2048


def _ln_kernel(x_ref, pos_ref, gamma_ref, beta_ref, o_ref):
    x = x_ref[...] + pos_ref[...][None, :, :]
    h = x.shape[-1]
    mean = jnp.sum(x, axis=-1, keepdims=True) * (1.0 / h)
    meansq = jnp.sum(x * x, axis=-1, keepdims=True) * (1.0 / h)
    var = meansq - mean * mean
    a = jax.lax.rsqrt(var + _EPS)
    o_ref[...] = x * (a * gamma_ref[...]) + (beta_ref[...] - (mean * a) * gamma_ref[...])


def kernel(concat_embeddings, concat_type, pos_table, gamma, beta):
    del concat_type
    B, S, H = concat_embeddings.shape
    bs = min(_BS, S)
    grid = (S // bs, B)

    return pl.pallas_call(
        _ln_kernel,
        grid=grid,
        in_specs=[
            pl.BlockSpec((1, bs, H), lambda s, b: (b, s, 0)),
            pl.BlockSpec((bs, H), lambda s, b: (s, 0)),
            pl.BlockSpec((H,), lambda s, b: (0,)),
            pl.BlockSpec((H,), lambda s, b: (0,)),
        ],
        out_specs=pl.BlockSpec((1, bs, H), lambda s, b: (b, s, 0)),
        out_shape=jax.ShapeDtypeStruct((B, S, H), concat_embeddings.dtype),
        compiler_params=pltpu.CompilerParams(
            dimension_semantics=("parallel", "parallel"),
            vmem_limit_bytes=128 * 1024 * 1024),
    )(concat_embeddings, pos_table, gamma, beta)


# speedup vs baseline: 1.0815x; 1.0815x over previous
import jax
import jax.numpy as jnp
from jax.experimental import pallas as pl
from jax.experimental.pallas import tpu as pltpu

_EPS = 1e-12
_BS = 2048


def _ln_kernel(x_ref, pos_ref, gamma_ref, beta_ref, o_ref):
    x = x_ref[...] + pos_ref[...][None, :, :]
    mean = jnp.mean(x, axis=-1, keepdims=True)
    cent = x - mean
    var = jnp.mean(cent * cent, axis=-1, keepdims=True)
    xhat = cent * jax.lax.rsqrt(var + _EPS)
    o_ref[...] = xhat * gamma_ref[...] + beta_ref[...]


def kernel(concat_embeddings, concat_type, pos_table, gamma, beta):
    del concat_type
    B, S, H = concat_embeddings.shape
    bs = 512
    grid = (S // bs,)

    return pl.pallas_call(
        _ln_kernel,
        grid=grid,
        in_specs=[
            pl.BlockSpec((B, bs, H), lambda s: (0, s, 0)),
            pl.BlockSpec((bs, H), lambda s: (s, 0)),
            pl.BlockSpec((H,), lambda s: (0,)),
            pl.BlockSpec((H,), lambda s: (0,)),
        ],
        out_specs=pl.BlockSpec((B, bs, H), lambda s: (0, s, 0)),
        out_shape=jax.ShapeDtypeStruct((B, S, H), concat_embeddings.dtype),
        compiler_params=pltpu.CompilerParams(
            dimension_semantics=("parallel",),
            vmem_limit_bytes=128 * 1024 * 1024),
    )(concat_embeddings, pos_table, gamma, beta)
